# baseline (speedup 1.0000x reference)
import dataclasses
import functools

import jax
import jax.numpy as jnp
from jax import lax
from jax.experimental import pallas as pl
from jax.experimental.pallas import tpu as pltpu
from jax.experimental.pallas import tpu_sc as plsc

N = 10000
E = 320000
C = 128
NC, NS = 2, 16
NW = NC * NS
CHUNK = 88
NCHT = 230
NCH0 = 139
NCH1 = NCHT - NCH0
EPW = NCHT * CHUNK // 2
E_PAD = 16 * NCHT * CHUNK
N_ACC = 10240
RPS = N_ACC // NS
E_PADT = 331776
EPW_DEG = E_PADT // NW
IDXC = 3456

_mesh = plsc.VectorSubcoreMesh(
    core_axis_name="c", subcore_axis_name="s", num_cores=NC, num_subcores=NS
)


def _deg_body(dst_hbm, zeros_hbm, out_hbm, hist_sh, hist_v, idx_v, idx2_v,
              tmp_v, red_v, dsem):
    cid = lax.axis_index("c")
    sid = lax.axis_index("s")
    wid = cid * NS + sid
    pltpu.sync_copy(zeros_hbm, hist_v)
    base = wid * EPW_DEG
    ones16 = jnp.full((16,), 1.0, jnp.float32)

    pltpu.sync_copy(dst_hbm.at[pl.ds(base, IDXC)], idx_v)
    pltpu.async_copy(dst_hbm.at[pl.ds(base + IDXC, IDXC)], idx2_v, dsem)

    @pl.loop(0, IDXC, step=16)
    def _(i):
        plsc.addupdate_scatter(hist_v, [idx_v[pl.ds(i, 16)]], ones16)

    pltpu.make_async_copy(dst_hbm.at[pl.ds(base + IDXC, IDXC)], idx2_v, dsem).wait()
    pltpu.async_copy(dst_hbm.at[pl.ds(base + 2 * IDXC, IDXC)], idx_v, dsem)

    @pl.loop(0, IDXC, step=16)
    def _(i):
        plsc.addupdate_scatter(hist_v, [idx2_v[pl.ds(i, 16)]], ones16)

    pltpu.make_async_copy(dst_hbm.at[pl.ds(base + 2 * IDXC, IDXC)], idx_v, dsem).wait()

    @pl.loop(0, IDXC, step=16)
    def _(i):
        plsc.addupdate_scatter(hist_v, [idx_v[pl.ds(i, 16)]], ones16)

    pltpu.sync_copy(hist_v, hist_sh.at[sid])
    plsc.subcore_barrier()
    for t in range(NS):
        pltpu.sync_copy(hist_sh.at[t, pl.ds(sid * RPS, RPS)], tmp_v.at[t])

    @pl.loop(0, RPS, step=16)
    def _(i):
        s = tmp_v[0, pl.ds(i, 16)]
        for t in range(1, NS):
            s = s + tmp_v[t, pl.ds(i, 16)]
        red_v[pl.ds(i, 16)] = s

    pltpu.sync_copy(red_v, out_hbm.at[cid, pl.ds(sid * RPS, RPS)])


_deg_cp = pltpu.CompilerParams()
if "needs_layout_passes" in pltpu.CompilerParams.__dataclass_fields__:
    _deg_cp = dataclasses.replace(_deg_cp, needs_layout_passes=False)

_deg_call = pl.kernel(
    _deg_body,
    out_type=jax.ShapeDtypeStruct((NC, N_ACC), jnp.float32),
    mesh=_mesh,
    compiler_params=_deg_cp,
    scratch_types=[
        pltpu.VMEM_SHARED((NS, N_ACC), jnp.float32),
        pltpu.VMEM((N_ACC,), jnp.float32),
        pltpu.VMEM((IDXC,), jnp.int32),
        pltpu.VMEM((IDXC,), jnp.int32),
        pltpu.VMEM((NS, RPS), jnp.float32),
        pltpu.VMEM((RPS,), jnp.float32),
        pltpu.SemaphoreType.DMA,
    ],
)


def _mm_body(x_ref, w_ref, deg_ref, hs_ref):
    h = jnp.dot(x_ref[...], w_ref[...], preferred_element_type=jnp.float32)
    deg = deg_ref[0, :, 0:1] + deg_ref[1, :, 0:1] + 1.0
    dis = lax.rsqrt(jnp.maximum(deg, 1e-12))
    hs_ref[...] = h * dis


_MB = 1000


def _mm_call(x, w, degp):
    return pl.pallas_call(
        _mm_body,
        out_shape=jax.ShapeDtypeStruct((N, C), jnp.float32),
        grid=(N // _MB,),
        in_specs=[
            pl.BlockSpec((_MB, C), lambda i: (i, 0)),
            pl.BlockSpec((C, C), lambda i: (0, 0)),
            pl.BlockSpec((NC, _MB, 1), lambda i: (0, i, 0)),
        ],
        out_specs=pl.BlockSpec((_MB, C), lambda i: (i, 0)),
    )(x, w, degp)


_U = 4


def _agg_body(hs_hbm, src_hbm, dst_hbm, zeros_hbm, out_hbm, acc_sh,
              src0, dst0, rows0, src1, dst1, rows1,
              src2, dst2, rows2, src3, dst3, rows3,
              sem0, sem1, sem2, sem3):
    srcs = [src0, src1, src2, src3]
    dsts = [dst0, dst1, dst2, dst3]
    rows = [rows0, rows1, rows2, rows3]
    sems = [sem0, sem1, sem2, sem3]
    cid = lax.axis_index("c")
    sid = lax.axis_index("s")
    pltpu.sync_copy(
        zeros_hbm.at[pl.ds(sid * RPS, RPS)], acc_sh.at[pl.ds(sid * RPS, RPS)]
    )
    plsc.subcore_barrier()
    nch = jnp.where(cid == 0, NCH0, NCH1)
    base = (cid * NS * NCH0 + sid * nch) * CHUNK
    ngroups = (nch - (_U - 1)) // _U

    for b in range(_U - 1):
        off = base + b * CHUNK
        pltpu.sync_copy(src_hbm.at[pl.ds(off, CHUNK)], srcs[b])
        pltpu.sync_copy(dst_hbm.at[pl.ds(off, CHUNK)], dsts[b])
        pltpu.async_copy(hs_hbm.at[srcs[b]], rows[b], sems[b])

    @pl.loop(0, ngroups)
    def _(j):
        for b in range(_U):
            kc = _U * j + b + _U - 1
            bc = (b + _U - 1) % _U
            offc = base + kc * CHUNK
            pltpu.sync_copy(src_hbm.at[pl.ds(offc, CHUNK)], srcs[bc])
            pltpu.sync_copy(dst_hbm.at[pl.ds(offc, CHUNK)], dsts[bc])
            pltpu.async_copy(hs_hbm.at[srcs[bc]], rows[bc], sems[bc])
            pltpu.make_async_copy(hs_hbm.at[srcs[b]], rows[b], sems[b]).wait()
            pltpu.sync_copy(rows[b], acc_sh.at[dsts[b]], add=True)

    for b in range(_U - 1):
        pltpu.make_async_copy(hs_hbm.at[srcs[b]], rows[b], sems[b]).wait()
        pltpu.sync_copy(rows[b], acc_sh.at[dsts[b]], add=True)

    plsc.subcore_barrier()
    pltpu.sync_copy(
        acc_sh.at[pl.ds(sid * RPS, RPS)],
        out_hbm.at[cid, pl.ds(sid * RPS, RPS)],
    )


_agg_call = pl.kernel(
    _agg_body,
    out_type=jax.ShapeDtypeStruct((NC, N_ACC, C), jnp.float32),
    mesh=_mesh,
    scratch_types=[
        pltpu.VMEM_SHARED((N_ACC, C), jnp.float32),
    ] + [
        t
        for _ in range(_U)
        for t in (
            pltpu.VMEM((CHUNK,), jnp.int32),
            pltpu.VMEM((CHUNK,), jnp.int32),
            pltpu.VMEM((CHUNK, C), jnp.float32),
        )
    ] + [pltpu.SemaphoreType.DMA] * _U,
)


def _epi_body(acc_ref, hs_ref, deg_ref, b_ref, out_ref):
    deg = deg_ref[0, :, 0:1] + deg_ref[1, :, 0:1] + 1.0
    dis = lax.rsqrt(jnp.maximum(deg, 1e-12))
    s = acc_ref[0] + acc_ref[1] + hs_ref[...]
    out_ref[...] = s * dis + b_ref[...]


def _epi_call(accp, hs, degp, b2):
    return pl.pallas_call(
        _epi_body,
        out_shape=jax.ShapeDtypeStruct((N, C), jnp.float32),
        grid=(N // _MB,),
        in_specs=[
            pl.BlockSpec((NC, _MB, C), lambda i: (0, i, 0)),
            pl.BlockSpec((_MB, C), lambda i: (i, 0)),
            pl.BlockSpec((NC, _MB, 1), lambda i: (0, i, 0)),
            pl.BlockSpec((1, C), lambda i: (0, 0)),
        ],
        out_specs=pl.BlockSpec((_MB, C), lambda i: (i, 0)),
    )(accp, hs, degp, b2)


def kernel(x, edge_index, W, b):
    src = edge_index[0].astype(jnp.int32)
    dst = edge_index[1].astype(jnp.int32)
    pad = E_PADT - E
    src_p = jnp.concatenate([src, jnp.zeros((pad,), jnp.int32)])
    pad_dst = N + (jnp.arange(pad, dtype=jnp.int32) % (N_ACC - N))
    dst_p = jnp.concatenate([dst, pad_dst])
    zeros_hist = jnp.zeros((N_ACC,), jnp.float32)
    zeros_acc = jnp.zeros((N_ACC, C), jnp.float32)

    degp = _deg_call(dst_p, zeros_hist)
    degp3 = degp.reshape(NC, N_ACC, 1)
    hs = _mm_call(x, W, degp3)
    accp = _agg_call(hs, src_p, dst_p, zeros_acc)
    out = _epi_call(accp, hs, degp3, b.reshape(1, C))
    return out


# speedup vs baseline: 1.2682x; 1.2682x over previous
import dataclasses

import jax
import jax.numpy as jnp
from jax import lax
from jax.experimental import pallas as pl
from jax.experimental.pallas import tpu as pltpu
from jax.experimental.pallas import tpu_sc as plsc

N = 10000
E = 320000
C = 128
NC, NS = 2, 16
NW = NC * NS
CHUNK = 80
NCHT = E // (NS * CHUNK)
NCH0 = 163
NCH1 = NCHT - NCH0
N_ACC = 10240
RPS = N_ACC // NS
EPW_DEG = E // NW
IDXC = 2000
ZR = 80

_mesh = plsc.VectorSubcoreMesh(
    core_axis_name="c", subcore_axis_name="s", num_cores=NC, num_subcores=NS
)

_sc_cp = pltpu.CompilerParams()
if "needs_layout_passes" in pltpu.CompilerParams.__dataclass_fields__:
    _sc_cp = dataclasses.replace(_sc_cp, needs_layout_passes=False)


def _deg_body(dst_hbm, out_hbm, hist_sh, hist_v, idx_v, idx2_v,
              tmp_v, red_v, dsem):
    cid = lax.axis_index("c")
    sid = lax.axis_index("s")
    wid = cid * NS + sid
    zeros16 = jnp.zeros((16,), jnp.float32)
    ones16 = jnp.full((16,), 1.0, jnp.float32)

    @pl.loop(0, N_ACC, step=16)
    def _(i):
        hist_v[pl.ds(i, 16)] = zeros16

    base = wid * EPW_DEG
    nchunk = EPW_DEG // IDXC

    pltpu.sync_copy(dst_hbm.at[pl.ds(base, IDXC)], idx_v)

    @pl.loop(0, (nchunk - 1) // 2)
    def _(j):
        off1 = base + (2 * j + 1) * IDXC
        pltpu.async_copy(dst_hbm.at[pl.ds(off1, IDXC)], idx2_v, dsem)

        @pl.loop(0, IDXC, step=16)
        def _(i):
            plsc.addupdate_scatter(hist_v, [idx_v[pl.ds(i, 16)]], ones16)

        pltpu.make_async_copy(dst_hbm.at[pl.ds(off1, IDXC)], idx2_v, dsem).wait()
        off2 = base + (2 * j + 2) * IDXC
        pltpu.async_copy(dst_hbm.at[pl.ds(off2, IDXC)], idx_v, dsem)

        @pl.loop(0, IDXC, step=16)
        def _(i):
            plsc.addupdate_scatter(hist_v, [idx2_v[pl.ds(i, 16)]], ones16)

        pltpu.make_async_copy(dst_hbm.at[pl.ds(off2, IDXC)], idx_v, dsem).wait()

    @pl.loop(0, IDXC, step=16)
    def _(i):
        plsc.addupdate_scatter(hist_v, [idx_v[pl.ds(i, 16)]], ones16)

    pltpu.sync_copy(hist_v, hist_sh.at[sid])
    plsc.subcore_barrier()
    for t in range(NS):
        pltpu.sync_copy(hist_sh.at[t, pl.ds(sid * RPS, RPS)], tmp_v.at[t])

    @pl.loop(0, RPS, step=16)
    def _(i):
        s = tmp_v[0, pl.ds(i, 16)]
        for t in range(1, NS):
            s = s + tmp_v[t, pl.ds(i, 16)]
        red_v[pl.ds(i, 16)] = s

    pltpu.sync_copy(red_v, out_hbm.at[cid, pl.ds(sid * RPS, RPS)])


_deg_call = pl.kernel(
    _deg_body,
    out_type=jax.ShapeDtypeStruct((NC, N_ACC), jnp.float32),
    mesh=_mesh,
    compiler_params=_sc_cp,
    scratch_types=[
        pltpu.VMEM_SHARED((NS, N_ACC), jnp.float32),
        pltpu.VMEM((N_ACC,), jnp.float32),
        pltpu.VMEM((IDXC,), jnp.int32),
        pltpu.VMEM((IDXC,), jnp.int32),
        pltpu.VMEM((NS, RPS), jnp.float32),
        pltpu.VMEM((RPS,), jnp.float32),
        pltpu.SemaphoreType.DMA,
    ],
)


def _mm_body(x_ref, w_ref, deg_ref, hs_ref):
    h = jnp.dot(x_ref[...], w_ref[...], preferred_element_type=jnp.float32)
    deg = deg_ref[0, :, 0:1] + deg_ref[1, :, 0:1] + 1.0
    dis = lax.rsqrt(jnp.maximum(deg, 1e-12))
    hs_ref[...] = h * dis


_MB = 1000


def _mm_call(x, w, degp):
    return pl.pallas_call(
        _mm_body,
        out_shape=jax.ShapeDtypeStruct((N, C), jnp.float32),
        grid=(N // _MB,),
        in_specs=[
            pl.BlockSpec((_MB, C), lambda i: (i, 0)),
            pl.BlockSpec((C, C), lambda i: (0, 0)),
            pl.BlockSpec((NC, _MB, 1), lambda i: (0, i, 0)),
        ],
        out_specs=pl.BlockSpec((_MB, C), lambda i: (i, 0)),
    )(x, w, degp)


_U = 4


def _agg_body(hs_hbm, src_hbm, dst_hbm, out_hbm, acc_sh,
              src0, dst0, rows0, src1, dst1, rows1,
              src2, dst2, rows2, src3, dst3, rows3,
              sem0, sem1, sem2, sem3):
    srcs = [src0, src1, src2, src3]
    dsts = [dst0, dst1, dst2, dst3]
    rows = [rows0, rows1, rows2, rows3]
    sems = [sem0, sem1, sem2, sem3]
    cid = lax.axis_index("c")
    sid = lax.axis_index("s")
    zeros16 = jnp.zeros((16,), jnp.float32)

    @pl.loop(0, CHUNK)
    def _(r):
        @pl.loop(0, C, step=16)
        def _(c):
            rows0[r, pl.ds(c, 16)] = zeros16

    for z in range(RPS // ZR):
        pltpu.sync_copy(rows0, acc_sh.at[pl.ds(sid * RPS + z * ZR, ZR)])
    plsc.subcore_barrier()

    nch = jnp.where(cid == 0, NCH0, NCH1)
    base = (cid * NS * NCH0 + sid * nch) * CHUNK
    ngroups = (nch - (_U - 1)) // _U

    for b in range(_U - 1):
        off = base + b * CHUNK
        pltpu.sync_copy(src_hbm.at[pl.ds(off, CHUNK)], srcs[b])
        pltpu.sync_copy(dst_hbm.at[pl.ds(off, CHUNK)], dsts[b])
        pltpu.async_copy(hs_hbm.at[srcs[b]], rows[b], sems[b])

    @pl.loop(0, ngroups)
    def _(j):
        for b in range(_U):
            kc = _U * j + b + _U - 1
            bc = (b + _U - 1) % _U
            offc = base + kc * CHUNK
            pltpu.sync_copy(src_hbm.at[pl.ds(offc, CHUNK)], srcs[bc])
            pltpu.sync_copy(dst_hbm.at[pl.ds(offc, CHUNK)], dsts[bc])
            pltpu.async_copy(hs_hbm.at[srcs[bc]], rows[bc], sems[bc])
            pltpu.make_async_copy(hs_hbm.at[srcs[b]], rows[b], sems[b]).wait()
            pltpu.sync_copy(rows[b], acc_sh.at[dsts[b]], add=True)

    for b in range(_U - 1):
        pltpu.make_async_copy(hs_hbm.at[srcs[b]], rows[b], sems[b]).wait()
        pltpu.sync_copy(rows[b], acc_sh.at[dsts[b]], add=True)

    plsc.subcore_barrier()
    pltpu.sync_copy(
        acc_sh.at[pl.ds(sid * RPS, RPS)],
        out_hbm.at[cid, pl.ds(sid * RPS, RPS)],
    )


_agg_call = pl.kernel(
    _agg_body,
    out_type=jax.ShapeDtypeStruct((NC, N_ACC, C), jnp.float32),
    mesh=_mesh,
    compiler_params=_sc_cp,
    scratch_types=[
        pltpu.VMEM_SHARED((N_ACC, C), jnp.float32),
    ] + [
        t
        for _ in range(_U)
        for t in (
            pltpu.VMEM((CHUNK,), jnp.int32),
            pltpu.VMEM((CHUNK,), jnp.int32),
            pltpu.VMEM((CHUNK, C), jnp.float32),
        )
    ] + [pltpu.SemaphoreType.DMA] * _U,
)


def _epi_body(acc_ref, hs_ref, deg_ref, b_ref, out_ref):
    deg = deg_ref[0, :, 0:1] + deg_ref[1, :, 0:1] + 1.0
    dis = lax.rsqrt(jnp.maximum(deg, 1e-12))
    s = acc_ref[0] + acc_ref[1] + hs_ref[...]
    out_ref[...] = s * dis + b_ref[...]


def _epi_call(accp, hs, degp, b2):
    return pl.pallas_call(
        _epi_body,
        out_shape=jax.ShapeDtypeStruct((N, C), jnp.float32),
        grid=(N // _MB,),
        in_specs=[
            pl.BlockSpec((NC, _MB, C), lambda i: (0, i, 0)),
            pl.BlockSpec((_MB, C), lambda i: (i, 0)),
            pl.BlockSpec((NC, _MB, 1), lambda i: (0, i, 0)),
            pl.BlockSpec((1, C), lambda i: (0, 0)),
        ],
        out_specs=pl.BlockSpec((_MB, C), lambda i: (i, 0)),
    )(accp, hs, degp, b2)


def kernel(x, edge_index, W, b):
    src = edge_index[0].astype(jnp.int32)
    dst = edge_index[1].astype(jnp.int32)
    degp = _deg_call(dst)
    degp3 = degp.reshape(NC, N_ACC, 1)
    hs = _mm_call(x, W, degp3)
    accp = _agg_call(hs, src, dst)
    out = _epi_call(accp, hs, degp3, b.reshape(1, C))
    return out


# speedup vs baseline: 1.4998x; 1.1826x over previous
import dataclasses

import jax
import jax.numpy as jnp
from jax import lax
from jax.experimental import pallas as pl
from jax.experimental.pallas import tpu as pltpu
from jax.experimental.pallas import tpu_sc as plsc

N = 10000
E = 320000
C = 128
NC, NS = 2, 16
NW = NC * NS
CHUNK = 80
NCHT = E // (NS * CHUNK)
NCH0 = 127
NCH1 = NCHT - NCH0
N_ACC = 10240
RPS = N_ACC // NS
EPW_DEG = E // NW
IDXC = 2000
ZR = 80

_mesh = plsc.VectorSubcoreMesh(
    core_axis_name="c", subcore_axis_name="s", num_cores=NC, num_subcores=NS
)

_sc_cp = pltpu.CompilerParams()
if "needs_layout_passes" in pltpu.CompilerParams.__dataclass_fields__:
    _sc_cp = dataclasses.replace(_sc_cp, needs_layout_passes=False)


def _deg_body(dst_hbm, out_hbm, hist_sh, hist_v, idx_v, idx2_v,
              tmp_v, red_v, dsem):
    cid = lax.axis_index("c")
    sid = lax.axis_index("s")
    wid = cid * NS + sid
    zeros16 = jnp.zeros((16,), jnp.float32)
    ones16 = jnp.full((16,), 1.0, jnp.float32)

    @pl.loop(0, N_ACC, step=16)
    def _(i):
        hist_v[pl.ds(i, 16)] = zeros16

    base = wid * EPW_DEG
    nchunk = EPW_DEG // IDXC

    pltpu.sync_copy(dst_hbm.at[pl.ds(base, IDXC)], idx_v)

    @pl.loop(0, (nchunk - 1) // 2)
    def _(j):
        off1 = base + (2 * j + 1) * IDXC
        pltpu.async_copy(dst_hbm.at[pl.ds(off1, IDXC)], idx2_v, dsem)

        @pl.loop(0, IDXC, step=16)
        def _(i):
            plsc.addupdate_scatter(hist_v, [idx_v[pl.ds(i, 16)]], ones16)

        pltpu.make_async_copy(dst_hbm.at[pl.ds(off1, IDXC)], idx2_v, dsem).wait()
        off2 = base + (2 * j + 2) * IDXC
        pltpu.async_copy(dst_hbm.at[pl.ds(off2, IDXC)], idx_v, dsem)

        @pl.loop(0, IDXC, step=16)
        def _(i):
            plsc.addupdate_scatter(hist_v, [idx2_v[pl.ds(i, 16)]], ones16)

        pltpu.make_async_copy(dst_hbm.at[pl.ds(off2, IDXC)], idx_v, dsem).wait()

    @pl.loop(0, IDXC, step=16)
    def _(i):
        plsc.addupdate_scatter(hist_v, [idx_v[pl.ds(i, 16)]], ones16)

    pltpu.sync_copy(hist_v, hist_sh.at[sid])
    plsc.subcore_barrier()
    for t in range(NS):
        pltpu.sync_copy(hist_sh.at[t, pl.ds(sid * RPS, RPS)], tmp_v.at[t])

    @pl.loop(0, RPS, step=16)
    def _(i):
        s = tmp_v[0, pl.ds(i, 16)]
        for t in range(1, NS):
            s = s + tmp_v[t, pl.ds(i, 16)]
        red_v[pl.ds(i, 16)] = s

    pltpu.sync_copy(red_v, out_hbm.at[cid, pl.ds(sid * RPS, RPS)])


_deg_call = pl.kernel(
    _deg_body,
    out_type=jax.ShapeDtypeStruct((NC, N_ACC), jnp.float32),
    mesh=_mesh,
    compiler_params=_sc_cp,
    scratch_types=[
        pltpu.VMEM_SHARED((NS, N_ACC), jnp.float32),
        pltpu.VMEM((N_ACC,), jnp.float32),
        pltpu.VMEM((IDXC,), jnp.int32),
        pltpu.VMEM((IDXC,), jnp.int32),
        pltpu.VMEM((NS, RPS), jnp.float32),
        pltpu.VMEM((RPS,), jnp.float32),
        pltpu.SemaphoreType.DMA,
    ],
)


def _mm_body(x_ref, w_ref, deg_ref, hs_ref):
    h = jnp.dot(x_ref[...], w_ref[...], preferred_element_type=jnp.float32)
    deg = deg_ref[0, :, 0:1] + deg_ref[1, :, 0:1] + 1.0
    dis = lax.rsqrt(jnp.maximum(deg, 1e-12))
    hs_ref[...] = h * dis


_MB = 1000


def _mm_call(x, w, degp):
    return pl.pallas_call(
        _mm_body,
        out_shape=jax.ShapeDtypeStruct((N, C), jnp.float32),
        grid=(N // _MB,),
        in_specs=[
            pl.BlockSpec((_MB, C), lambda i: (i, 0)),
            pl.BlockSpec((C, C), lambda i: (0, 0)),
            pl.BlockSpec((NC, _MB, 1), lambda i: (0, i, 0)),
        ],
        out_specs=pl.BlockSpec((_MB, C), lambda i: (i, 0)),
    )(x, w, degp)


_U = 4


def _agg_body(hs_hbm, src_hbm, dst_hbm, out_hbm, acc_sh,
              src0, dst0, rows0, src1, dst1, rows1,
              src2, dst2, rows2, src3, dst3, rows3,
              sem0, sem1, sem2, sem3):
    srcs = [src0, src1, src2, src3]
    dsts = [dst0, dst1, dst2, dst3]
    rows = [rows0, rows1, rows2, rows3]
    sems = [sem0, sem1, sem2, sem3]
    cid = lax.axis_index("c")
    sid = lax.axis_index("s")
    zeros16 = jnp.zeros((16,), jnp.float32)

    @pl.loop(0, CHUNK)
    def _(r):
        @pl.loop(0, C, step=16)
        def _(c):
            rows0[r, pl.ds(c, 16)] = zeros16

    for z in range(RPS // ZR):
        pltpu.sync_copy(rows0, acc_sh.at[pl.ds(sid * RPS + z * ZR, ZR)])
    plsc.subcore_barrier()

    nch = jnp.where(cid == 0, NCH0, NCH1)
    base = (cid * NS * NCH0 + sid * nch) * CHUNK
    ngroups = (nch - (_U - 1)) // _U

    for b in range(_U - 1):
        off = base + b * CHUNK
        pltpu.sync_copy(src_hbm.at[pl.ds(off, CHUNK)], srcs[b])
        pltpu.sync_copy(dst_hbm.at[pl.ds(off, CHUNK)], dsts[b])
        pltpu.async_copy(hs_hbm.at[srcs[b]], rows[b], sems[b])

    @pl.loop(0, ngroups)
    def _(j):
        for b in range(_U):
            kc = _U * j + b + _U - 1
            bc = (b + _U - 1) % _U
            offc = base + kc * CHUNK
            pltpu.sync_copy(src_hbm.at[pl.ds(offc, CHUNK)], srcs[bc])
            pltpu.sync_copy(dst_hbm.at[pl.ds(offc, CHUNK)], dsts[bc])
            pltpu.async_copy(hs_hbm.at[srcs[bc]], rows[bc], sems[bc])
            pltpu.make_async_copy(hs_hbm.at[srcs[b]], rows[b], sems[b]).wait()
            pltpu.sync_copy(rows[b], acc_sh.at[dsts[b]], add=True)

    for b in range(_U - 1):
        pltpu.make_async_copy(hs_hbm.at[srcs[b]], rows[b], sems[b]).wait()
        pltpu.sync_copy(rows[b], acc_sh.at[dsts[b]], add=True)

    plsc.subcore_barrier()
    pltpu.sync_copy(
        acc_sh.at[pl.ds(sid * RPS, RPS)],
        out_hbm.at[cid, pl.ds(sid * RPS, RPS)],
    )


_agg_call = pl.kernel(
    _agg_body,
    out_type=jax.ShapeDtypeStruct((NC, N_ACC, C), jnp.float32),
    mesh=_mesh,
    compiler_params=_sc_cp,
    scratch_types=[
        pltpu.VMEM_SHARED((N_ACC, C), jnp.float32),
    ] + [
        t
        for _ in range(_U)
        for t in (
            pltpu.VMEM((CHUNK,), jnp.int32),
            pltpu.VMEM((CHUNK,), jnp.int32),
            pltpu.VMEM((CHUNK, C), jnp.float32),
        )
    ] + [pltpu.SemaphoreType.DMA] * _U,
)


def _epi_body(acc_ref, hs_ref, deg_ref, b_ref, out_ref):
    deg = deg_ref[0, :, 0:1] + deg_ref[1, :, 0:1] + 1.0
    dis = lax.rsqrt(jnp.maximum(deg, 1e-12))
    s = acc_ref[0] + acc_ref[1] + hs_ref[...]
    out_ref[...] = s * dis + b_ref[...]


def _epi_call(accp, hs, degp, b2):
    return pl.pallas_call(
        _epi_body,
        out_shape=jax.ShapeDtypeStruct((N, C), jnp.float32),
        grid=(N // _MB,),
        in_specs=[
            pl.BlockSpec((NC, _MB, C), lambda i: (0, i, 0)),
            pl.BlockSpec((_MB, C), lambda i: (i, 0)),
            pl.BlockSpec((NC, _MB, 1), lambda i: (0, i, 0)),
            pl.BlockSpec((1, C), lambda i: (0, 0)),
        ],
        out_specs=pl.BlockSpec((_MB, C), lambda i: (i, 0)),
    )(accp, hs, degp, b2)


def kernel(x, edge_index, W, b):
    src = edge_index[0].astype(jnp.int32)
    dst = edge_index[1].astype(jnp.int32)
    degp = _deg_call(dst)
    degp3 = degp.reshape(NC, N_ACC, 1)
    hs = _mm_call(x, W, degp3)
    accp = _agg_call(hs, src, dst)
    out = _epi_call(accp, hs, degp3, b.reshape(1, C))
    return out


# speedup vs baseline: 1.5110x; 1.0075x over previous
import dataclasses

import jax
import jax.numpy as jnp
from jax import lax
from jax.experimental import pallas as pl
from jax.experimental.pallas import tpu as pltpu
from jax.experimental.pallas import tpu_sc as plsc

N = 10000
E = 320000
C = 128
NC, NS = 2, 16
NW = NC * NS
CHUNK = 80
NCHT = E // (NS * CHUNK)
NCH0 = 125
NCH1 = NCHT - NCH0
N_ACC = 10240
RPS = N_ACC // NS
EPW_DEG = E // NW
IDXC = 2000
ZR = 80

_mesh = plsc.VectorSubcoreMesh(
    core_axis_name="c", subcore_axis_name="s", num_cores=NC, num_subcores=NS
)

_sc_cp = pltpu.CompilerParams()
if "needs_layout_passes" in pltpu.CompilerParams.__dataclass_fields__:
    _sc_cp = dataclasses.replace(_sc_cp, needs_layout_passes=False)


def _deg_body(dst_hbm, out_hbm, hist_sh, hist_v, idx_v, idx2_v,
              tmp_v, red_v, dsem):
    cid = lax.axis_index("c")
    sid = lax.axis_index("s")
    wid = cid * NS + sid
    zeros16 = jnp.zeros((16,), jnp.float32)
    ones16 = jnp.full((16,), 1.0, jnp.float32)

    @pl.loop(0, N_ACC, step=16)
    def _(i):
        hist_v[pl.ds(i, 16)] = zeros16

    base = wid * EPW_DEG
    nchunk = EPW_DEG // IDXC

    pltpu.sync_copy(dst_hbm.at[pl.ds(base, IDXC)], idx_v)

    @pl.loop(0, (nchunk - 1) // 2)
    def _(j):
        off1 = base + (2 * j + 1) * IDXC
        pltpu.async_copy(dst_hbm.at[pl.ds(off1, IDXC)], idx2_v, dsem)

        @pl.loop(0, IDXC, step=16)
        def _(i):
            plsc.addupdate_scatter(hist_v, [idx_v[pl.ds(i, 16)]], ones16)

        pltpu.make_async_copy(dst_hbm.at[pl.ds(off1, IDXC)], idx2_v, dsem).wait()
        off2 = base + (2 * j + 2) * IDXC
        pltpu.async_copy(dst_hbm.at[pl.ds(off2, IDXC)], idx_v, dsem)

        @pl.loop(0, IDXC, step=16)
        def _(i):
            plsc.addupdate_scatter(hist_v, [idx2_v[pl.ds(i, 16)]], ones16)

        pltpu.make_async_copy(dst_hbm.at[pl.ds(off2, IDXC)], idx_v, dsem).wait()

    @pl.loop(0, IDXC, step=16)
    def _(i):
        plsc.addupdate_scatter(hist_v, [idx_v[pl.ds(i, 16)]], ones16)

    pltpu.sync_copy(hist_v, hist_sh.at[sid])
    plsc.subcore_barrier()
    for t in range(NS):
        pltpu.sync_copy(hist_sh.at[t, pl.ds(sid * RPS, RPS)], tmp_v.at[t])

    @pl.loop(0, RPS, step=16)
    def _(i):
        s = tmp_v[0, pl.ds(i, 16)]
        for t in range(1, NS):
            s = s + tmp_v[t, pl.ds(i, 16)]
        red_v[pl.ds(i, 16)] = s

    pltpu.sync_copy(red_v, out_hbm.at[cid, pl.ds(sid * RPS, RPS)])


_deg_call = pl.kernel(
    _deg_body,
    out_type=jax.ShapeDtypeStruct((NC, N_ACC), jnp.float32),
    mesh=_mesh,
    compiler_params=_sc_cp,
    scratch_types=[
        pltpu.VMEM_SHARED((NS, N_ACC), jnp.float32),
        pltpu.VMEM((N_ACC,), jnp.float32),
        pltpu.VMEM((IDXC,), jnp.int32),
        pltpu.VMEM((IDXC,), jnp.int32),
        pltpu.VMEM((NS, RPS), jnp.float32),
        pltpu.VMEM((RPS,), jnp.float32),
        pltpu.SemaphoreType.DMA,
    ],
)


def _mm_body(x_ref, w_ref, deg_ref, hs_ref):
    h = jnp.dot(x_ref[...], w_ref[...], preferred_element_type=jnp.float32)
    deg = deg_ref[0, :, 0:1] + deg_ref[1, :, 0:1] + 1.0
    dis = lax.rsqrt(jnp.maximum(deg, 1e-12))
    hs_ref[...] = h * dis


_MB = 1000


def _mm_call(x, w, degp):
    return pl.pallas_call(
        _mm_body,
        out_shape=jax.ShapeDtypeStruct((N, C), jnp.float32),
        grid=(N // _MB,),
        in_specs=[
            pl.BlockSpec((_MB, C), lambda i: (i, 0)),
            pl.BlockSpec((C, C), lambda i: (0, 0)),
            pl.BlockSpec((NC, _MB, 1), lambda i: (0, i, 0)),
        ],
        out_specs=pl.BlockSpec((_MB, C), lambda i: (i, 0)),
    )(x, w, degp)


_U = 2


def _agg_body(hs_hbm, src_hbm, dst_hbm, out_hbm, acc_sh,
              src0, dst0, rows0, src1, dst1, rows1,
              src2, dst2, rows2, src3, dst3, rows3,
              sem0, sem1, sem2, sem3):
    srcs = [src0, src1, src2, src3][:_U]
    dsts = [dst0, dst1, dst2, dst3][:_U]
    rows = [rows0, rows1, rows2, rows3][:_U]
    sems = [sem0, sem1, sem2, sem3][:_U]
    cid = lax.axis_index("c")
    sid = lax.axis_index("s")
    zeros16 = jnp.zeros((16,), jnp.float32)

    @pl.loop(0, CHUNK)
    def _(r):
        @pl.loop(0, C, step=16)
        def _(c):
            rows0[r, pl.ds(c, 16)] = zeros16

    for z in range(RPS // ZR):
        pltpu.sync_copy(rows0, acc_sh.at[pl.ds(sid * RPS + z * ZR, ZR)])
    plsc.subcore_barrier()

    nch = jnp.where(cid == 0, NCH0, NCH1)
    base = (cid * NS * NCH0 + sid * nch) * CHUNK
    ngroups = (nch - (_U - 1)) // _U

    for b in range(_U - 1):
        off = base + b * CHUNK
        pltpu.sync_copy(src_hbm.at[pl.ds(off, CHUNK)], srcs[b])
        pltpu.sync_copy(dst_hbm.at[pl.ds(off, CHUNK)], dsts[b])
        pltpu.async_copy(hs_hbm.at[srcs[b]], rows[b], sems[b])

    @pl.loop(0, ngroups)
    def _(j):
        for b in range(_U):
            kc = _U * j + b + _U - 1
            bc = (b + _U - 1) % _U
            offc = base + kc * CHUNK
            pltpu.sync_copy(src_hbm.at[pl.ds(offc, CHUNK)], srcs[bc])
            pltpu.sync_copy(dst_hbm.at[pl.ds(offc, CHUNK)], dsts[bc])
            pltpu.async_copy(hs_hbm.at[srcs[bc]], rows[bc], sems[bc])
            pltpu.make_async_copy(hs_hbm.at[srcs[b]], rows[b], sems[b]).wait()
            pltpu.sync_copy(rows[b], acc_sh.at[dsts[b]], add=True)

    for b in range(_U - 1):
        pltpu.make_async_copy(hs_hbm.at[srcs[b]], rows[b], sems[b]).wait()
        pltpu.sync_copy(rows[b], acc_sh.at[dsts[b]], add=True)

    plsc.subcore_barrier()
    pltpu.sync_copy(
        acc_sh.at[pl.ds(sid * RPS, RPS)],
        out_hbm.at[cid, pl.ds(sid * RPS, RPS)],
    )


_agg_call = pl.kernel(
    _agg_body,
    out_type=jax.ShapeDtypeStruct((NC, N_ACC, C), jnp.float32),
    mesh=_mesh,
    compiler_params=_sc_cp,
    scratch_types=[
        pltpu.VMEM_SHARED((N_ACC, C), jnp.float32),
    ] + [
        t
        for _ in range(4)
        for t in (
            pltpu.VMEM((CHUNK,), jnp.int32),
            pltpu.VMEM((CHUNK,), jnp.int32),
            pltpu.VMEM((CHUNK, C), jnp.float32),
        )
    ] + [pltpu.SemaphoreType.DMA] * 4,
)


def _epi_body(acc_ref, hs_ref, deg_ref, b_ref, out_ref):
    deg = deg_ref[0, :, 0:1] + deg_ref[1, :, 0:1] + 1.0
    dis = lax.rsqrt(jnp.maximum(deg, 1e-12))
    s = acc_ref[0] + acc_ref[1] + hs_ref[...]
    out_ref[...] = s * dis + b_ref[...]


def _epi_call(accp, hs, degp, b2):
    return pl.pallas_call(
        _epi_body,
        out_shape=jax.ShapeDtypeStruct((N, C), jnp.float32),
        grid=(N // _MB,),
        in_specs=[
            pl.BlockSpec((NC, _MB, C), lambda i: (0, i, 0)),
            pl.BlockSpec((_MB, C), lambda i: (i, 0)),
            pl.BlockSpec((NC, _MB, 1), lambda i: (0, i, 0)),
            pl.BlockSpec((1, C), lambda i: (0, 0)),
        ],
        out_specs=pl.BlockSpec((_MB, C), lambda i: (i, 0)),
    )(accp, hs, degp, b2)


def kernel(x, edge_index, W, b):
    src = edge_index[0].astype(jnp.int32)
    dst = edge_index[1].astype(jnp.int32)
    degp = _deg_call(dst)
    degp3 = degp.reshape(NC, N_ACC, 1)
    hs = _mm_call(x, W, degp3)
    accp = _agg_call(hs, src, dst)
    out = _epi_call(accp, hs, degp3, b.reshape(1, C))
    return out


# speedup vs baseline: 1.5159x; 1.0032x over previous
import dataclasses

import jax
import jax.numpy as jnp
from jax import lax
from jax.experimental import pallas as pl
from jax.experimental.pallas import tpu as pltpu
from jax.experimental.pallas import tpu_sc as plsc

N = 10000
E = 320000
C = 128
NC, NS = 2, 16
NW = NC * NS
CHUNK = 80
NCHT = E // (NS * CHUNK)
NCH0 = 125
NCH1 = NCHT - NCH0
N_ACC = 10240
RPS = N_ACC // NS
EPW_DEG = E // NW
IDXC = 2000
ZR = 80

_mesh = plsc.VectorSubcoreMesh(
    core_axis_name="c", subcore_axis_name="s", num_cores=NC, num_subcores=NS
)

_sc_cp = pltpu.CompilerParams()
if "needs_layout_passes" in pltpu.CompilerParams.__dataclass_fields__:
    _sc_cp = dataclasses.replace(_sc_cp, needs_layout_passes=False)


def _deg_body(dst_hbm, out_hbm, hist_sh, hist_v, idx_v, idx2_v,
              tmp_v, red_v, dsem):
    cid = lax.axis_index("c")
    sid = lax.axis_index("s")
    wid = cid * NS + sid
    zeros16 = jnp.zeros((16,), jnp.float32)
    ones16 = jnp.full((16,), 1.0, jnp.float32)

    @pl.loop(0, N_ACC, step=16)
    def _(i):
        hist_v[pl.ds(i, 16)] = zeros16

    base = wid * EPW_DEG
    nchunk = EPW_DEG // IDXC

    pltpu.sync_copy(dst_hbm.at[pl.ds(base, IDXC)], idx_v)

    @pl.loop(0, (nchunk - 1) // 2)
    def _(j):
        off1 = base + (2 * j + 1) * IDXC
        pltpu.async_copy(dst_hbm.at[pl.ds(off1, IDXC)], idx2_v, dsem)

        @pl.loop(0, IDXC, step=16)
        def _(i):
            plsc.addupdate_scatter(hist_v, [idx_v[pl.ds(i, 16)]], ones16)

        pltpu.make_async_copy(dst_hbm.at[pl.ds(off1, IDXC)], idx2_v, dsem).wait()
        off2 = base + (2 * j + 2) * IDXC
        pltpu.async_copy(dst_hbm.at[pl.ds(off2, IDXC)], idx_v, dsem)

        @pl.loop(0, IDXC, step=16)
        def _(i):
            plsc.addupdate_scatter(hist_v, [idx2_v[pl.ds(i, 16)]], ones16)

        pltpu.make_async_copy(dst_hbm.at[pl.ds(off2, IDXC)], idx_v, dsem).wait()

    @pl.loop(0, IDXC, step=16)
    def _(i):
        plsc.addupdate_scatter(hist_v, [idx_v[pl.ds(i, 16)]], ones16)

    pltpu.sync_copy(hist_v, hist_sh.at[sid])
    plsc.subcore_barrier()
    for t in range(NS):
        pltpu.sync_copy(hist_sh.at[t, pl.ds(sid * RPS, RPS)], tmp_v.at[t])

    @pl.loop(0, RPS, step=16)
    def _(i):
        s = tmp_v[0, pl.ds(i, 16)]
        for t in range(1, NS):
            s = s + tmp_v[t, pl.ds(i, 16)]
        red_v[pl.ds(i, 16)] = s

    pltpu.sync_copy(red_v, out_hbm.at[cid, pl.ds(sid * RPS, RPS)])


_deg_call = pl.kernel(
    _deg_body,
    out_type=jax.ShapeDtypeStruct((NC, N_ACC), jnp.float32),
    mesh=_mesh,
    compiler_params=_sc_cp,
    scratch_types=[
        pltpu.VMEM_SHARED((NS, N_ACC), jnp.float32),
        pltpu.VMEM((N_ACC,), jnp.float32),
        pltpu.VMEM((IDXC,), jnp.int32),
        pltpu.VMEM((IDXC,), jnp.int32),
        pltpu.VMEM((NS, RPS), jnp.float32),
        pltpu.VMEM((RPS,), jnp.float32),
        pltpu.SemaphoreType.DMA,
    ],
)


def _mm_body(x_ref, w_ref, deg_ref, hs_ref):
    h = jnp.dot(x_ref[...], w_ref[...], preferred_element_type=jnp.float32)
    deg = deg_ref[0, :, 0:1] + deg_ref[1, :, 0:1] + 1.0
    dis = lax.rsqrt(jnp.maximum(deg, 1e-12))
    hs_ref[...] = h * dis


_MB = 1000


def _mm_call(x, w, degp):
    return pl.pallas_call(
        _mm_body,
        out_shape=jax.ShapeDtypeStruct((N, C), jnp.float32),
        grid=(N // _MB,),
        in_specs=[
            pl.BlockSpec((_MB, C), lambda i: (i, 0)),
            pl.BlockSpec((C, C), lambda i: (0, 0)),
            pl.BlockSpec((NC, _MB, 1), lambda i: (0, i, 0)),
        ],
        out_specs=pl.BlockSpec((_MB, C), lambda i: (i, 0)),
    )(x, w, degp)


_U = 3


def _agg_body(hs_hbm, src_hbm, dst_hbm, out_hbm, acc_sh,
              src0, dst0, rows0, src1, dst1, rows1,
              src2, dst2, rows2, src3, dst3, rows3,
              sem0, sem1, sem2, sem3):
    srcs = [src0, src1, src2, src3][:_U]
    dsts = [dst0, dst1, dst2, dst3][:_U]
    rows = [rows0, rows1, rows2, rows3][:_U]
    sems = [sem0, sem1, sem2, sem3][:_U]
    cid = lax.axis_index("c")
    sid = lax.axis_index("s")
    zeros16 = jnp.zeros((16,), jnp.float32)

    @pl.loop(0, CHUNK)
    def _(r):
        @pl.loop(0, C, step=16)
        def _(c):
            rows0[r, pl.ds(c, 16)] = zeros16

    for z in range(RPS // ZR):
        pltpu.sync_copy(rows0, acc_sh.at[pl.ds(sid * RPS + z * ZR, ZR)])
    plsc.subcore_barrier()

    nch = jnp.where(cid == 0, NCH0, NCH1)
    base = (cid * NS * NCH0 + sid * nch) * CHUNK
    ngroups = (nch - (_U - 1)) // _U

    for b in range(_U - 1):
        off = base + b * CHUNK
        pltpu.sync_copy(src_hbm.at[pl.ds(off, CHUNK)], srcs[b])
        pltpu.sync_copy(dst_hbm.at[pl.ds(off, CHUNK)], dsts[b])
        pltpu.async_copy(hs_hbm.at[srcs[b]], rows[b], sems[b])

    @pl.loop(0, ngroups)
    def _(j):
        for b in range(_U):
            kc = _U * j + b + _U - 1
            bc = (b + _U - 1) % _U
            offc = base + kc * CHUNK
            pltpu.sync_copy(src_hbm.at[pl.ds(offc, CHUNK)], srcs[bc])
            pltpu.sync_copy(dst_hbm.at[pl.ds(offc, CHUNK)], dsts[bc])
            pltpu.async_copy(hs_hbm.at[srcs[bc]], rows[bc], sems[bc])
            pltpu.make_async_copy(hs_hbm.at[srcs[b]], rows[b], sems[b]).wait()
            pltpu.sync_copy(rows[b], acc_sh.at[dsts[b]], add=True)

    for b in range(_U - 1):
        pltpu.make_async_copy(hs_hbm.at[srcs[b]], rows[b], sems[b]).wait()
        pltpu.sync_copy(rows[b], acc_sh.at[dsts[b]], add=True)

    plsc.subcore_barrier()
    pltpu.sync_copy(
        acc_sh.at[pl.ds(sid * RPS, RPS)],
        out_hbm.at[cid, pl.ds(sid * RPS, RPS)],
    )


_agg_call = pl.kernel(
    _agg_body,
    out_type=jax.ShapeDtypeStruct((NC, N_ACC, C), jnp.float32),
    mesh=_mesh,
    compiler_params=_sc_cp,
    scratch_types=[
        pltpu.VMEM_SHARED((N_ACC, C), jnp.float32),
    ] + [
        t
        for _ in range(4)
        for t in (
            pltpu.VMEM((CHUNK,), jnp.int32),
            pltpu.VMEM((CHUNK,), jnp.int32),
            pltpu.VMEM((CHUNK, C), jnp.float32),
        )
    ] + [pltpu.SemaphoreType.DMA] * 4,
)


def _epi_body(acc_ref, hs_ref, deg_ref, b_ref, out_ref):
    deg = deg_ref[0, :, 0:1] + deg_ref[1, :, 0:1] + 1.0
    dis = lax.rsqrt(jnp.maximum(deg, 1e-12))
    s = acc_ref[0] + acc_ref[1] + hs_ref[...]
    out_ref[...] = s * dis + b_ref[...]


def _epi_call(accp, hs, degp, b2):
    return pl.pallas_call(
        _epi_body,
        out_shape=jax.ShapeDtypeStruct((N, C), jnp.float32),
        grid=(N // _MB,),
        in_specs=[
            pl.BlockSpec((NC, _MB, C), lambda i: (0, i, 0)),
            pl.BlockSpec((_MB, C), lambda i: (i, 0)),
            pl.BlockSpec((NC, _MB, 1), lambda i: (0, i, 0)),
            pl.BlockSpec((1, C), lambda i: (0, 0)),
        ],
        out_specs=pl.BlockSpec((_MB, C), lambda i: (i, 0)),
    )(accp, hs, degp, b2)


def kernel(x, edge_index, W, b):
    src = edge_index[0].astype(jnp.int32)
    dst = edge_index[1].astype(jnp.int32)
    degp = _deg_call(dst)
    degp3 = degp.reshape(NC, N_ACC, 1)
    hs = _mm_call(x, W, degp3)
    accp = _agg_call(hs, src, dst)
    out = _epi_call(accp, hs, degp3, b.reshape(1, C))
    return out
